# Initial kernel scaffold; baseline (speedup 1.0000x reference)
import jax
import jax.numpy as jnp
from jax.experimental import pallas as pl


def kernel(x, gate_W, gate_b, W1, b1, W2, b2):
    raise NotImplementedError("write your pallas kernel here")


# speedup vs baseline: 4.3555x; 4.3555x over previous
import functools

import jax
import jax.numpy as jnp
from jax.experimental import pallas as pl

NUM_EXPERTS = 8
TOP_K = 2
INPUT_DIM = 2048
OUTPUT_DIM = 2048
HIDDEN = 128

BT = 256


def _moe_body(x_ref, gw_ref, gb_ref, w1_ref, b1_ref, w2_ref, b2_ref, out_ref):
    xb = x_ref[...]
    logits = jnp.dot(xb, gw_ref[...], preferred_element_type=jnp.float32)
    logits = logits + gb_ref[...]
    m = jnp.max(logits, axis=-1, keepdims=True)
    ex = jnp.exp(logits - m)
    w = ex / jnp.sum(ex, axis=-1, keepdims=True)

    iota = jax.lax.broadcasted_iota(jnp.int32, (BT, NUM_EXPERTS), 1)
    big = jnp.int32(NUM_EXPERTS)
    m1 = jnp.max(w, axis=-1, keepdims=True)
    i1 = jnp.min(jnp.where(w == m1, iota, big), axis=-1, keepdims=True)
    mask1 = iota == i1
    w_rem = jnp.where(mask1, -1.0, w)
    m2 = jnp.max(w_rem, axis=-1, keepdims=True)
    i2 = jnp.min(jnp.where(w_rem == m2, iota, big), axis=-1, keepdims=True)
    mask2 = iota == i2
    c = jnp.where(mask1 | mask2, w, 0.0)

    acc = jnp.dot(c, b2_ref[...], preferred_element_type=jnp.float32)
    for e in range(NUM_EXPERTS):
        h = jnp.dot(xb, w1_ref[e], preferred_element_type=jnp.float32)
        h = jnp.maximum(h + b1_ref[e][None, :], 0.0)
        hw = h * c[:, e][:, None]
        acc = acc + jnp.dot(hw, w2_ref[e], preferred_element_type=jnp.float32)
    out_ref[...] = acc


def kernel(x, gate_W, gate_b, W1, b1, W2, b2):
    B, S, d = x.shape
    T = B * S
    x_flat = x.reshape(T, d)
    gb2 = gate_b.reshape(1, NUM_EXPERTS)

    grid = (T // BT,)
    out = pl.pallas_call(
        _moe_body,
        grid=grid,
        in_specs=[
            pl.BlockSpec((BT, d), lambda i: (i, 0)),
            pl.BlockSpec((d, NUM_EXPERTS), lambda i: (0, 0)),
            pl.BlockSpec((1, NUM_EXPERTS), lambda i: (0, 0)),
            pl.BlockSpec((NUM_EXPERTS, d, HIDDEN), lambda i: (0, 0, 0)),
            pl.BlockSpec((NUM_EXPERTS, HIDDEN), lambda i: (0, 0)),
            pl.BlockSpec((NUM_EXPERTS, HIDDEN, OUTPUT_DIM), lambda i: (0, 0, 0)),
            pl.BlockSpec((NUM_EXPERTS, OUTPUT_DIM), lambda i: (0, 0)),
        ],
        out_specs=pl.BlockSpec((BT, OUTPUT_DIM), lambda i: (i, 0)),
        out_shape=jax.ShapeDtypeStruct((T, OUTPUT_DIM), jnp.float32),
    )(x_flat, gate_W, gb2, W1, b1, W2, b2)
    return out.reshape(B, S, OUTPUT_DIM)


# speedup vs baseline: 9.2780x; 2.1302x over previous
import jax
import jax.numpy as jnp
from jax.experimental import pallas as pl

NUM_EXPERTS = 8
TOP_K = 2
INPUT_DIM = 2048
OUTPUT_DIM = 2048
HIDDEN = 128
EH = NUM_EXPERTS * HIDDEN

BT = 256


def _moe_body(x_ref, gw_ref, gb_ref, w1_ref, b1_ref, w2_ref, b2_ref, out_ref):
    xb = x_ref[...]
    logits = jnp.dot(xb, gw_ref[...], preferred_element_type=jnp.float32)
    logits = logits + gb_ref[...]
    m = jnp.max(logits, axis=-1, keepdims=True)
    ex = jnp.exp(logits - m)
    w = ex / jnp.sum(ex, axis=-1, keepdims=True)

    iota = jax.lax.broadcasted_iota(jnp.int32, (BT, NUM_EXPERTS), 1)
    big = jnp.int32(NUM_EXPERTS)
    m1 = jnp.max(w, axis=-1, keepdims=True)
    i1 = jnp.min(jnp.where(w == m1, iota, big), axis=-1, keepdims=True)
    mask1 = iota == i1
    w_rem = jnp.where(mask1, -1.0, w)
    m2 = jnp.max(w_rem, axis=-1, keepdims=True)
    i2 = jnp.min(jnp.where(w_rem == m2, iota, big), axis=-1, keepdims=True)
    mask2 = iota == i2
    c = jnp.where(mask1 | mask2, w, 0.0)

    xb16 = xb.astype(jnp.bfloat16)
    h = jnp.dot(xb16, w1_ref[...], preferred_element_type=jnp.float32)
    h = jnp.maximum(h + b1_ref[...], 0.0)
    cexp = jnp.broadcast_to(c[:, :, None], (BT, NUM_EXPERTS, HIDDEN))
    cexp = cexp.reshape(BT, EH)
    hc = (h * cexp).astype(jnp.bfloat16)
    acc = jnp.dot(hc, w2_ref[...], preferred_element_type=jnp.float32)
    acc = acc + jnp.dot(c, b2_ref[...], preferred_element_type=jnp.float32)
    out_ref[...] = acc


def kernel(x, gate_W, gate_b, W1, b1, W2, b2):
    B, S, d = x.shape
    T = B * S
    x_flat = x.reshape(T, d)
    gb2 = gate_b.reshape(1, NUM_EXPERTS)
    w1cat = W1.transpose(1, 0, 2).reshape(d, EH).astype(jnp.bfloat16)
    b1cat = b1.reshape(1, EH)
    w2cat = W2.reshape(EH, OUTPUT_DIM).astype(jnp.bfloat16)

    grid = (T // BT,)
    out = pl.pallas_call(
        _moe_body,
        grid=grid,
        in_specs=[
            pl.BlockSpec((BT, d), lambda i: (i, 0)),
            pl.BlockSpec((d, NUM_EXPERTS), lambda i: (0, 0)),
            pl.BlockSpec((1, NUM_EXPERTS), lambda i: (0, 0)),
            pl.BlockSpec((d, EH), lambda i: (0, 0)),
            pl.BlockSpec((1, EH), lambda i: (0, 0)),
            pl.BlockSpec((EH, OUTPUT_DIM), lambda i: (0, 0)),
            pl.BlockSpec((NUM_EXPERTS, OUTPUT_DIM), lambda i: (0, 0)),
        ],
        out_specs=pl.BlockSpec((BT, OUTPUT_DIM), lambda i: (i, 0)),
        out_shape=jax.ShapeDtypeStruct((T, OUTPUT_DIM), jnp.float32),
    )(x_flat, gate_W, gb2, w1cat, b1cat, w2cat, b2)
    return out.reshape(B, S, OUTPUT_DIM)


# speedup vs baseline: 9.4588x; 1.0195x over previous
import jax
import jax.numpy as jnp
from jax.experimental import pallas as pl
from jax.experimental.pallas import tpu as pltpu

NUM_EXPERTS = 8
TOP_K = 2
INPUT_DIM = 2048
OUTPUT_DIM = 2048
HIDDEN = 128
EH = NUM_EXPERTS * HIDDEN

BT = 512


def _moe_body(x_ref, gw_ref, gb_ref, w1_ref, b1_ref, w2_ref, b2_ref, out_ref):
    xb = x_ref[...]
    logits = jnp.dot(xb, gw_ref[...], preferred_element_type=jnp.float32)
    logits = logits + gb_ref[...]
    m = jnp.max(logits, axis=-1, keepdims=True)
    ex = jnp.exp(logits - m)
    w = ex / jnp.sum(ex, axis=-1, keepdims=True)

    iota = jax.lax.broadcasted_iota(jnp.int32, (BT, NUM_EXPERTS), 1)
    big = jnp.int32(NUM_EXPERTS)
    m1 = jnp.max(w, axis=-1, keepdims=True)
    i1 = jnp.min(jnp.where(w == m1, iota, big), axis=-1, keepdims=True)
    mask1 = iota == i1
    w_rem = jnp.where(mask1, -1.0, w)
    m2 = jnp.max(w_rem, axis=-1, keepdims=True)
    i2 = jnp.min(jnp.where(w_rem == m2, iota, big), axis=-1, keepdims=True)
    mask2 = iota == i2
    c = jnp.where(mask1 | mask2, w, 0.0)

    xb16 = xb.astype(jnp.bfloat16)
    h = jnp.dot(xb16, w1_ref[...], preferred_element_type=jnp.float32)
    h = jnp.maximum(h + b1_ref[...], 0.0)
    cexp = jnp.broadcast_to(c[:, :, None], (BT, NUM_EXPERTS, HIDDEN))
    cexp = cexp.reshape(BT, EH)
    hc = (h * cexp).astype(jnp.bfloat16)
    acc = jnp.dot(hc, w2_ref[...], preferred_element_type=jnp.float32)
    acc = acc + jnp.dot(c, b2_ref[...], preferred_element_type=jnp.float32)
    out_ref[...] = acc


def kernel(x, gate_W, gate_b, W1, b1, W2, b2):
    B, S, d = x.shape
    T = B * S
    x_flat = x.reshape(T, d)
    gb2 = gate_b.reshape(1, NUM_EXPERTS)
    w1cat = W1.transpose(1, 0, 2).reshape(d, EH).astype(jnp.bfloat16)
    b1cat = b1.reshape(1, EH)
    w2cat = W2.reshape(EH, OUTPUT_DIM).astype(jnp.bfloat16)

    grid = (T // BT,)
    out = pl.pallas_call(
        _moe_body,
        grid=grid,
        in_specs=[
            pl.BlockSpec((BT, d), lambda i: (i, 0)),
            pl.BlockSpec((d, NUM_EXPERTS), lambda i: (0, 0)),
            pl.BlockSpec((1, NUM_EXPERTS), lambda i: (0, 0)),
            pl.BlockSpec((d, EH), lambda i: (0, 0)),
            pl.BlockSpec((1, EH), lambda i: (0, 0)),
            pl.BlockSpec((EH, OUTPUT_DIM), lambda i: (0, 0)),
            pl.BlockSpec((NUM_EXPERTS, OUTPUT_DIM), lambda i: (0, 0)),
        ],
        out_specs=pl.BlockSpec((BT, OUTPUT_DIM), lambda i: (i, 0)),
        out_shape=jax.ShapeDtypeStruct((T, OUTPUT_DIM), jnp.float32),
        compiler_params=pltpu.CompilerParams(
            dimension_semantics=("parallel",),
        ),
    )(x_flat, gate_W, gb2, w1cat, b1cat, w2cat, b2)
    return out.reshape(B, S, OUTPUT_DIM)


# speedup vs baseline: 10.3402x; 1.0932x over previous
import jax
import jax.numpy as jnp
from jax.experimental import pallas as pl
from jax.experimental.pallas import tpu as pltpu

NUM_EXPERTS = 8
TOP_K = 2
INPUT_DIM = 2048
OUTPUT_DIM = 2048
HIDDEN = 128
EH = NUM_EXPERTS * HIDDEN

BT = 512


def _moe_body(x_ref, gw_ref, gb_ref, w1_ref, b1_ref, w2_ref, b2_ref, out_ref,
              w1s, w2s):
    @pl.when(pl.program_id(0) == 0)
    def _init():
        for e in range(NUM_EXPERTS):
            w1s[:, e * HIDDEN:(e + 1) * HIDDEN] = w1_ref[e].astype(jnp.bfloat16)
        w2s[...] = w2_ref[...].astype(jnp.bfloat16)

    xb = x_ref[...]
    logits = jnp.dot(xb, gw_ref[...], preferred_element_type=jnp.float32)
    logits = logits + gb_ref[...]
    m = jnp.max(logits, axis=-1, keepdims=True)
    ex = jnp.exp(logits - m)
    w = ex / jnp.sum(ex, axis=-1, keepdims=True)

    iota = jax.lax.broadcasted_iota(jnp.int32, (BT, NUM_EXPERTS), 1)
    big = jnp.int32(NUM_EXPERTS)
    m1 = jnp.max(w, axis=-1, keepdims=True)
    i1 = jnp.min(jnp.where(w == m1, iota, big), axis=-1, keepdims=True)
    mask1 = iota == i1
    w_rem = jnp.where(mask1, -1.0, w)
    m2 = jnp.max(w_rem, axis=-1, keepdims=True)
    i2 = jnp.min(jnp.where(w_rem == m2, iota, big), axis=-1, keepdims=True)
    mask2 = iota == i2
    c = jnp.where(mask1 | mask2, w, 0.0)

    xb16 = xb.astype(jnp.bfloat16)
    h = jnp.dot(xb16, w1s[...], preferred_element_type=jnp.float32)
    h = jnp.maximum(h + b1_ref[...], 0.0)
    cexp = jnp.broadcast_to(c[:, :, None], (BT, NUM_EXPERTS, HIDDEN))
    cexp = cexp.reshape(BT, EH)
    hc = (h * cexp).astype(jnp.bfloat16)
    acc = jnp.dot(hc, w2s[...], preferred_element_type=jnp.float32)
    acc = acc + jnp.dot(c, b2_ref[...], preferred_element_type=jnp.float32)
    out_ref[...] = acc


def kernel(x, gate_W, gate_b, W1, b1, W2, b2):
    B, S, d = x.shape
    T = B * S
    x_flat = x.reshape(T, d)
    gb2 = gate_b.reshape(1, NUM_EXPERTS)
    b1cat = b1.reshape(1, EH)
    w2r = W2.reshape(EH, OUTPUT_DIM)

    grid = (T // BT,)
    out = pl.pallas_call(
        _moe_body,
        grid=grid,
        in_specs=[
            pl.BlockSpec((BT, d), lambda i: (i, 0)),
            pl.BlockSpec((d, NUM_EXPERTS), lambda i: (0, 0)),
            pl.BlockSpec((1, NUM_EXPERTS), lambda i: (0, 0)),
            pl.BlockSpec((NUM_EXPERTS, d, HIDDEN), lambda i: (0, 0, 0)),
            pl.BlockSpec((1, EH), lambda i: (0, 0)),
            pl.BlockSpec((EH, OUTPUT_DIM), lambda i: (0, 0)),
            pl.BlockSpec((NUM_EXPERTS, OUTPUT_DIM), lambda i: (0, 0)),
        ],
        out_specs=pl.BlockSpec((BT, OUTPUT_DIM), lambda i: (i, 0)),
        out_shape=jax.ShapeDtypeStruct((T, OUTPUT_DIM), jnp.float32),
        scratch_shapes=[
            pltpu.VMEM((INPUT_DIM, EH), jnp.bfloat16),
            pltpu.VMEM((EH, OUTPUT_DIM), jnp.bfloat16),
        ],
    )(x_flat, gate_W, gb2, W1, b1cat, w2r, b2)
    return out.reshape(B, S, OUTPUT_DIM)
